# baseline (speedup 1.0000x reference)
import functools

import jax
import jax.numpy as jnp
from jax import lax
from jax.experimental import pallas as pl
from jax.experimental.pallas import tpu as pltpu
from jax.experimental.pallas import tpu_sc as plsc


def _gather_rows(n_total: int, d: int, chunk: int):
    info = plsc.get_sparse_core_info()
    nc, ns = info.num_cores, info.num_subcores
    nw = nc * ns
    per_w = n_total // nw
    n_chunks = per_w // chunk
    mesh = plsc.VectorSubcoreMesh(core_axis_name="c", subcore_axis_name="s")

    @functools.partial(
        pl.kernel,
        mesh=mesh,
        compiler_params=pltpu.CompilerParams(use_tc_tiling_on_sc=False),
        out_type=jax.ShapeDtypeStruct((n_total, d), jnp.float32),
        scratch_types=[
            pltpu.VMEM((chunk,), jnp.int32),
            pltpu.VMEM((chunk, d), jnp.float32),
            pltpu.SemaphoreType.DMA,
        ],
    )
    def k(idx_hbm, table_hbm, out_hbm, idx_v, rows_v, sem):
        wid = lax.axis_index("s") * nc + lax.axis_index("c")
        base = pl.multiple_of(wid * per_w, 8)

        def body(i, _):
            off = pl.multiple_of(base + i * chunk, 8)
            pltpu.sync_copy(idx_hbm.at[pl.ds(off, chunk)], idx_v)
            pltpu.async_copy(table_hbm.at[idx_v], rows_v, sem).wait()
            pltpu.sync_copy(rows_v, out_hbm.at[pl.ds(off, chunk)])
            return 0

        lax.fori_loop(0, n_chunks, body, 0)

    return k


def kernel(x, table):
    b, h = x.shape
    v, d = table.shape
    n = b * h
    xf = x.reshape(n).astype(jnp.int32)
    out = _gather_rows(n, d, 512)(xf, table)
    return out.reshape(b, h, d)


# speedup vs baseline: 1.0405x; 1.0405x over previous
import functools

import jax
import jax.numpy as jnp
from jax import lax
from jax.experimental import pallas as pl
from jax.experimental.pallas import tpu as pltpu
from jax.experimental.pallas import tpu_sc as plsc


def _gather_rows(n_total: int, d: int, chunk: int, nbuf: int):
    info = plsc.get_sparse_core_info()
    nc, ns = info.num_cores, info.num_subcores
    nw = nc * ns
    per_w = n_total // nw
    n_chunks = per_w // chunk
    n_groups = n_chunks // nbuf
    assert per_w % chunk == 0 and n_chunks % nbuf == 0 and chunk % 8 == 0
    mesh = plsc.VectorSubcoreMesh(core_axis_name="c", subcore_axis_name="s")

    @functools.partial(
        pl.kernel,
        mesh=mesh,
        compiler_params=pltpu.CompilerParams(use_tc_tiling_on_sc=False),
        out_type=jax.ShapeDtypeStruct((n_total, d), jnp.float32),
        scratch_types=[
            pltpu.VMEM((per_w,), jnp.int32),
            pltpu.VMEM((nbuf, chunk, d), jnp.float32),
            pltpu.SemaphoreType.DMA((nbuf,)),
            pltpu.SemaphoreType.DMA((nbuf,)),
            pltpu.SemaphoreType.DMA,
        ],
    )
    def k(idx_hbm, table_hbm, out_hbm, idx_v, rows_v, gsem, ssem, isem):
        wid = lax.axis_index("s") * nc + lax.axis_index("c")
        base = pl.multiple_of(wid * per_w, 8)
        pltpu.async_copy(idx_hbm.at[pl.ds(base, per_w)], idx_v, isem).wait()

        def gather_start(k_idx, b):
            return pltpu.async_copy(
                table_hbm.at[idx_v.at[pl.ds(k_idx * chunk, chunk)]],
                rows_v.at[b], gsem.at[b])

        def gather_wait(k_idx, b):
            pltpu.make_async_copy(
                table_hbm.at[idx_v.at[pl.ds(k_idx * chunk, chunk)]],
                rows_v.at[b], gsem.at[b]).wait()

        def scatter_start(k_idx, b):
            off = pl.multiple_of(base + k_idx * chunk, 8)
            return pltpu.async_copy(
                rows_v.at[b], out_hbm.at[pl.ds(off, chunk)], ssem.at[b])

        def scatter_wait(k_idx, b):
            off = pl.multiple_of(base + k_idx * chunk, 8)
            pltpu.make_async_copy(
                rows_v.at[b], out_hbm.at[pl.ds(off, chunk)], ssem.at[b]).wait()

        gather_start(0, 0)
        for j in range(1, nbuf):
            gather_start(j, j)
            gather_wait(j - 1, j - 1)
            scatter_start(j - 1, j - 1)

        def group(g, _):
            for b in range(nbuf):
                k_idx = g * nbuf + b
                bp = (b - 1) % nbuf
                scatter_wait(k_idx - nbuf, b)
                gather_start(k_idx, b)
                gather_wait(k_idx - 1, bp)
                scatter_start(k_idx - 1, bp)
            return 0

        lax.fori_loop(1, n_groups, group, 0)

        last = n_chunks - 1
        bl = last % nbuf
        gather_wait(last, bl)
        scatter_start(last, bl)
        for b in range(nbuf):
            scatter_wait(n_chunks - nbuf + b, b)

    return k


def kernel(x, table):
    b, h = x.shape
    v, d = table.shape
    n = b * h
    xf = x.reshape(n).astype(jnp.int32)
    out = _gather_rows(n, d, 320, 4)(xf, table)
    return out.reshape(b, h, d)


# speedup vs baseline: 1.2679x; 1.2186x over previous
import functools

import jax
import jax.numpy as jnp
from jax import lax
from jax.experimental import pallas as pl
from jax.experimental.pallas import tpu as pltpu
from jax.experimental.pallas import tpu_sc as plsc


def _gather_rows(n_total: int, d: int, dpad: int, chunk: int, nbuf: int):
    info = plsc.get_sparse_core_info()
    nc, ns = info.num_cores, info.num_subcores
    nw = nc * ns
    per_w = n_total // nw
    n_chunks = per_w // chunk
    n_groups = n_chunks // nbuf
    assert per_w % chunk == 0 and n_chunks % nbuf == 0 and chunk % 8 == 0
    mesh = plsc.VectorSubcoreMesh(core_axis_name="c", subcore_axis_name="s")

    @functools.partial(
        pl.kernel,
        mesh=mesh,
        compiler_params=pltpu.CompilerParams(use_tc_tiling_on_sc=True),
        out_type=jax.ShapeDtypeStruct((n_total, dpad), jnp.float32),
        scratch_types=[
            pltpu.VMEM((per_w,), jnp.int32),
            pltpu.VMEM((nbuf, chunk, dpad), jnp.float32),
            pltpu.SemaphoreType.DMA((nbuf,)),
            pltpu.SemaphoreType.DMA((nbuf,)),
            pltpu.SemaphoreType.DMA,
        ],
    )
    def k(idx_hbm, table_hbm, out_hbm, idx_v, rows_v, gsem, ssem, isem):
        wid = lax.axis_index("s") * nc + lax.axis_index("c")
        base = pl.multiple_of(wid * per_w, 8)
        pltpu.async_copy(idx_hbm.at[pl.ds(base, per_w)], idx_v, isem).wait()

        def gather_start(k_idx, b):
            return pltpu.async_copy(
                table_hbm.at[idx_v.at[pl.ds(k_idx * chunk, chunk)]],
                rows_v.at[b], gsem.at[b])

        def gather_wait(k_idx, b):
            pltpu.make_async_copy(
                table_hbm.at[idx_v.at[pl.ds(k_idx * chunk, chunk)]],
                rows_v.at[b], gsem.at[b]).wait()

        def scatter_start(k_idx, b):
            off = pl.multiple_of(base + k_idx * chunk, 8)
            return pltpu.async_copy(
                rows_v.at[b], out_hbm.at[pl.ds(off, chunk)], ssem.at[b])

        def scatter_wait(k_idx, b):
            off = pl.multiple_of(base + k_idx * chunk, 8)
            pltpu.make_async_copy(
                rows_v.at[b], out_hbm.at[pl.ds(off, chunk)], ssem.at[b]).wait()

        gather_start(0, 0)
        for j in range(1, nbuf):
            gather_start(j, j)
            gather_wait(j - 1, j - 1)
            scatter_start(j - 1, j - 1)

        def group(g, _):
            for b in range(nbuf):
                k_idx = g * nbuf + b
                bp = (b - 1) % nbuf
                scatter_wait(k_idx - nbuf, b)
                gather_start(k_idx, b)
                gather_wait(k_idx - 1, bp)
                scatter_start(k_idx - 1, bp)
            return 0

        lax.fori_loop(1, n_groups, group, 0)

        last = n_chunks - 1
        bl = last % nbuf
        gather_wait(last, bl)
        scatter_start(last, bl)
        for b in range(nbuf):
            scatter_wait(n_chunks - nbuf + b, b)

    return k


def kernel(x, table):
    b, h = x.shape
    v, d = table.shape
    n = b * h
    dpad = 128
    xf = x.reshape(n).astype(jnp.int32)
    tp = jnp.pad(table, ((0, 0), (0, dpad - d)))
    out = _gather_rows(n, d, dpad, 160, 4)(xf, tp)
    return out[:, :d].reshape(b, h, d)
